# Initial kernel scaffold; baseline (speedup 1.0000x reference)
import jax
import jax.numpy as jnp
from jax.experimental import pallas as pl


def kernel(x, edge_index, edge_attr, batch, We, be, Wpre, bpre, Wpost, bpost, Wlin, blin, gamma, beta, Wih0, Whh0, bih0, bhh0, Wih1, Whh1, bih1, bhh1):
    raise NotImplementedError("write your pallas kernel here")


# speedup vs baseline: 12.8864x; 12.8864x over previous
import functools

import jax
import jax.numpy as jnp
from jax import lax
from jax.experimental import pallas as pl
from jax.experimental.pallas import tpu as pltpu

_L = 2
_N = 10000
_E = 160000
_D = 128
_DE = 16
_T = 4
_FOUT = 32
_B = 64
_STEPS = 5
_TD = _T * _D

_NT = 2000
_ET = 2000


def _uv_body(h_ref, wc_ref, bu_ref, uv_ref):
    uv_ref[...] = (
        jnp.dot(h_ref[...], wc_ref[...], preferred_element_type=jnp.float32)
        + bu_ref[...]
    )


def _uv(h, wc, bu):
    return pl.pallas_call(
        _uv_body,
        grid=(_N // _NT,),
        in_specs=[
            pl.BlockSpec((_NT, _D), lambda i: (i, 0)),
            pl.BlockSpec((_D, 2 * _TD), lambda i: (0, 0)),
            pl.BlockSpec((1, 2 * _TD), lambda i: (0, 0)),
        ],
        out_specs=pl.BlockSpec((_NT, 2 * _TD), lambda i: (i, 0)),
        out_shape=jax.ShapeDtypeStruct((_N, 2 * _TD), jnp.float32),
    )(h, wc, bu)


def _w_body(ea_ref, wm_ref, w_ref):
    w_ref[...] = jnp.dot(
        ea_ref[...], wm_ref[...], preferred_element_type=jnp.float32
    )


def _edge_w(edge_attr, wm):
    return pl.pallas_call(
        _w_body,
        grid=(_E // _ET,),
        in_specs=[
            pl.BlockSpec((_ET, _DE), lambda i: (i, 0)),
            pl.BlockSpec((_DE, _TD), lambda i: (0, 0)),
        ],
        out_specs=pl.BlockSpec((_ET, _TD), lambda i: (i, 0)),
        out_shape=jax.ShapeDtypeStruct((_E, _TD), jnp.float32),
    )(edge_attr, wm)


def _post_body(x_ref, u_ref, sx_ref, se_ref, deg_ref, zmn_ref, zmx_ref,
               ws_ref, wm_ref, wpost_ref, bpost_ref, wlin_ref, blin_ref,
               out_ref, stats_ref, acc_ref):
    i = pl.program_id(0)
    x = x_ref[...]
    u = u_ref[...]
    deg = deg_ref[...]
    ssum = (
        deg * u
        + jnp.dot(sx_ref[...], ws_ref[...], preferred_element_type=jnp.float32)
        + jnp.dot(se_ref[...], wm_ref[...], preferred_element_type=jnp.float32)
    )
    mean = ssum / jnp.maximum(deg, 1.0)
    has = deg > 0.0
    mn = jnp.where(has, u + zmn_ref[...], 0.0)
    mx = jnp.where(has, u + zmx_ref[...], 0.0)
    wpost = wpost_ref[...]
    bpost = bpost_ref[...]
    ys = []
    for t in range(_T):
        sl = slice(t * _D, (t + 1) * _D)
        y = jnp.dot(x, wpost[t, 0:_D, :], preferred_element_type=jnp.float32)
        y = y + jnp.dot(mean[:, sl], wpost[t, _D:2 * _D, :],
                        preferred_element_type=jnp.float32)
        y = y + jnp.dot(mn[:, sl], wpost[t, 2 * _D:3 * _D, :],
                        preferred_element_type=jnp.float32)
        y = y + jnp.dot(mx[:, sl], wpost[t, 3 * _D:4 * _D, :],
                        preferred_element_type=jnp.float32)
        ys.append(y)
    out = jnp.concatenate(ys, axis=1) + bpost
    out = (
        jnp.dot(out, wlin_ref[...], preferred_element_type=jnp.float32)
        + blin_ref[...]
    )
    out_ref[...] = out

    @pl.when(i == 0)
    def _():
        acc_ref[...] = jnp.zeros_like(acc_ref)

    acc_ref[0:1, :] += jnp.sum(out, axis=0, keepdims=True)
    acc_ref[1:2, :] += jnp.sum(out * out, axis=0, keepdims=True)

    @pl.when(i == pl.num_programs(0) - 1)
    def _():
        stats_ref[...] = acc_ref[...]


def _post(x, u, sx, se, deg, zmn, zmx, ws, wm, wpost, bpost, wlin, blin):
    return pl.pallas_call(
        _post_body,
        grid=(_N // _NT,),
        in_specs=[
            pl.BlockSpec((_NT, _D), lambda i: (i, 0)),
            pl.BlockSpec((_NT, _TD), lambda i: (i, 0)),
            pl.BlockSpec((_NT, _D), lambda i: (i, 0)),
            pl.BlockSpec((_NT, _DE), lambda i: (i, 0)),
            pl.BlockSpec((_NT, 1), lambda i: (i, 0)),
            pl.BlockSpec((_NT, _TD), lambda i: (i, 0)),
            pl.BlockSpec((_NT, _TD), lambda i: (i, 0)),
            pl.BlockSpec((_D, _TD), lambda i: (0, 0)),
            pl.BlockSpec((_DE, _TD), lambda i: (0, 0)),
            pl.BlockSpec((_T, 4 * _D, _FOUT), lambda i: (0, 0, 0)),
            pl.BlockSpec((1, _D), lambda i: (0, 0)),
            pl.BlockSpec((_D, _D), lambda i: (0, 0)),
            pl.BlockSpec((1, _D), lambda i: (0, 0)),
        ],
        out_specs=[
            pl.BlockSpec((_NT, _D), lambda i: (i, 0)),
            pl.BlockSpec((8, _D), lambda i: (0, 0)),
        ],
        out_shape=[
            jax.ShapeDtypeStruct((_N, _D), jnp.float32),
            jax.ShapeDtypeStruct((8, _D), jnp.float32),
        ],
        scratch_shapes=[pltpu.VMEM((8, _D), jnp.float32)],
    )(x, u, sx, se, deg, zmn, zmx, ws, wm, wpost, bpost, wlin, blin)


def _bn_body(raw_ref, stats_ref, gamma_ref, beta_ref, out_ref):
    s = stats_ref[...]
    mu = s[0:1, :] / _N
    var = s[1:2, :] / _N - mu * mu
    rstd = lax.rsqrt(var + 1e-5)
    out = (raw_ref[...] - mu) * rstd * gamma_ref[...] + beta_ref[...]
    out_ref[...] = jnp.where(out >= 0.0, out, 0.01 * out)


def _bn(raw, stats, gamma, beta):
    return pl.pallas_call(
        _bn_body,
        grid=(_N // _NT,),
        in_specs=[
            pl.BlockSpec((_NT, _D), lambda i: (i, 0)),
            pl.BlockSpec((8, _D), lambda i: (0, 0)),
            pl.BlockSpec((1, _D), lambda i: (0, 0)),
            pl.BlockSpec((1, _D), lambda i: (0, 0)),
        ],
        out_specs=pl.BlockSpec((_NT, _D), lambda i: (i, 0)),
        out_shape=jax.ShapeDtypeStruct((_N, _D), jnp.float32),
    )(raw, stats, gamma, beta)


def _s2s_body(h0_ref, h1_ref, batch_ref, wih0_ref, whh0_ref, b0_ref,
              wih1_ref, whh1_ref, b1_ref, out_ref):
    x = jnp.maximum(h0_ref[...], h1_ref[...])
    bt = batch_ref[...]
    iota = lax.broadcasted_iota(jnp.int32, (_N, _B), 1)
    onehotb = bt == iota
    onehot = onehotb.astype(jnp.float32)

    q_star = jnp.zeros((_B, 2 * _D), jnp.float32)
    h0 = jnp.zeros((_B, _D), jnp.float32)
    c0 = jnp.zeros((_B, _D), jnp.float32)
    h1 = jnp.zeros((_B, _D), jnp.float32)
    c1 = jnp.zeros((_B, _D), jnp.float32)

    def cell(xin, h, c, wih, whh, b):
        g = (
            lax.dot_general(xin, wih, (((1,), (1,)), ((), ())),
                            preferred_element_type=jnp.float32)
            + lax.dot_general(h, whh, (((1,), (1,)), ((), ())),
                              preferred_element_type=jnp.float32)
            + b
        )
        i = jax.nn.sigmoid(g[:, 0:_D])
        f = jax.nn.sigmoid(g[:, _D:2 * _D])
        gg = jnp.tanh(g[:, 2 * _D:3 * _D])
        o = jax.nn.sigmoid(g[:, 3 * _D:4 * _D])
        c2 = f * c + i * gg
        return o * jnp.tanh(c2), c2

    for _ in range(_STEPS):
        h0, c0 = cell(q_star, h0, c0, wih0_ref[...], whh0_ref[...], b0_ref[...])
        h1, c1 = cell(h0, h1, c1, wih1_ref[...], whh1_ref[...], b1_ref[...])
        q = h1
        qb = jnp.dot(onehot, q, preferred_element_type=jnp.float32)
        e = jnp.sum(x * qb, axis=1, keepdims=True)
        m = jnp.max(jnp.where(onehotb, e, -jnp.inf), axis=0, keepdims=True)
        m = jnp.where(jnp.isfinite(m), m, 0.0)
        mb = jnp.sum(onehot * m, axis=1, keepdims=True)
        ex = jnp.exp(e - mb)
        s = jnp.sum(onehot * ex, axis=0, keepdims=True)
        sb = jnp.sum(onehot * s, axis=1, keepdims=True)
        a = ex / (sb + 1e-16)
        r = lax.dot_general(onehot * a, x, (((0,), (0,)), ((), ())),
                            preferred_element_type=jnp.float32)
        q_star = jnp.concatenate([q, r], axis=1)
    out_ref[...] = q_star


def _s2s(h0, h1, batch2d, wih0, whh0, b0, wih1, whh1, b1):
    return pl.pallas_call(
        _s2s_body,
        out_shape=jax.ShapeDtypeStruct((_B, 2 * _D), jnp.float32),
    )(h0, h1, batch2d, wih0, whh0, b0, wih1, whh1, b1)


@jax.jit
def kernel(x, edge_index, edge_attr, batch, We, be, Wpre, bpre, Wpost, bpost,
           Wlin, blin, gamma, beta, Wih0, Whh0, bih0, bhh0, Wih1, Whh1,
           bih1, bhh1):
    src = edge_index[0]
    dst = edge_index[1]
    ones = jnp.ones((_E,), jnp.float32)
    deg = jax.ops.segment_sum(ones, dst, num_segments=_N)
    deg2d = deg[:, None]
    se = jax.ops.segment_sum(edge_attr, dst, num_segments=_N)

    h = x
    layer_outs = []
    for l in range(_L):
        wd = jnp.transpose(Wpre[l, :, 0:_D, :], (1, 0, 2)).reshape(_D, _TD)
        ws = jnp.transpose(Wpre[l, :, _D:2 * _D, :], (1, 0, 2)).reshape(_D, _TD)
        wein = jnp.transpose(Wpre[l, :, 2 * _D:3 * _D, :], (1, 0, 2)).reshape(_D, _TD)
        wm = We[l] @ wein
        ub = bpre[l].reshape(_TD) + be[l] @ wein
        wc = jnp.concatenate([wd, ws], axis=1)
        bu = jnp.concatenate([ub, jnp.zeros((_TD,), jnp.float32)])[None, :]

        uv = _uv(h, wc, bu)
        u = uv[:, 0:_TD]
        v = uv[:, _TD:]
        w = _edge_w(edge_attr, wm)

        z = v[src] + w
        zmn = jax.ops.segment_min(z, dst, num_segments=_N)
        zmx = jax.ops.segment_max(z, dst, num_segments=_N)
        sx = jax.ops.segment_sum(h[src], dst, num_segments=_N)

        raw, stats = _post(
            h, u, sx, se, deg2d, zmn, zmx, ws, wm, Wpost[l],
            bpost[l].reshape(1, _T * _FOUT), Wlin[l], blin[l][None, :])
        h = _bn(raw, stats, gamma[l][None, :], beta[l][None, :])
        layer_outs.append(h)

    return _s2s(layer_outs[0], layer_outs[1], batch[:, None],
                Wih0, Whh0, (bih0 + bhh0)[None, :],
                Wih1, Whh1, (bih1 + bhh1)[None, :])


# speedup vs baseline: 14.3638x; 1.1146x over previous
import functools

import jax
import jax.numpy as jnp
from jax import lax
from jax.experimental import pallas as pl
from jax.experimental.pallas import tpu as pltpu

_L = 2
_N = 10000
_E = 160000
_D = 128
_DE = 16
_T = 4
_FOUT = 32
_B = 64
_STEPS = 5
_TD = _T * _D

_NT = 2000
_ET = 2000


def _uv_body(h_ref, wc_ref, bu_ref, uv_ref):
    uv_ref[...] = (
        jnp.dot(h_ref[...], wc_ref[...], preferred_element_type=jnp.float32)
        + bu_ref[...]
    )


def _uv(h, wc, bu):
    return pl.pallas_call(
        _uv_body,
        grid=(_N // _NT,),
        in_specs=[
            pl.BlockSpec((_NT, _D), lambda i: (i, 0)),
            pl.BlockSpec((_D, 2 * _TD), lambda i: (0, 0)),
            pl.BlockSpec((1, 2 * _TD), lambda i: (0, 0)),
        ],
        out_specs=pl.BlockSpec((_NT, 2 * _TD), lambda i: (i, 0)),
        out_shape=jax.ShapeDtypeStruct((_N, 2 * _TD), jnp.float32),
    )(h, wc, bu)


def _w_body(ea_ref, wm_ref, w_ref):
    w_ref[...] = jnp.dot(
        ea_ref[...], wm_ref[...], preferred_element_type=jnp.float32
    )


def _edge_w(edge_attr, wm):
    return pl.pallas_call(
        _w_body,
        grid=(_E // _ET,),
        in_specs=[
            pl.BlockSpec((_ET, _DE), lambda i: (i, 0)),
            pl.BlockSpec((_DE, _TD), lambda i: (0, 0)),
        ],
        out_specs=pl.BlockSpec((_ET, _TD), lambda i: (i, 0)),
        out_shape=jax.ShapeDtypeStruct((_E, _TD), jnp.float32),
    )(edge_attr, wm)


def _post_body(x_ref, u_ref, sx_ref, se_ref, deg_ref, zmn_ref, zmx_ref,
               ws_ref, wm_ref, wpost_ref, bpost_ref, wlin_ref, blin_ref,
               out_ref, stats_ref, acc_ref):
    i = pl.program_id(0)
    x = x_ref[...]
    u = u_ref[...]
    deg = deg_ref[...]
    ssum = (
        deg * u
        + jnp.dot(sx_ref[...], ws_ref[...], preferred_element_type=jnp.float32)
        + jnp.dot(se_ref[...], wm_ref[...], preferred_element_type=jnp.float32)
    )
    mean = ssum / jnp.maximum(deg, 1.0)
    has = deg > 0.0
    mn = jnp.where(has, u + zmn_ref[...], 0.0)
    mx = jnp.where(has, u + zmx_ref[...], 0.0)
    wpost = wpost_ref[...]
    bpost = bpost_ref[...]
    ys = []
    for t in range(_T):
        sl = slice(t * _D, (t + 1) * _D)
        y = jnp.dot(x, wpost[t, 0:_D, :], preferred_element_type=jnp.float32)
        y = y + jnp.dot(mean[:, sl], wpost[t, _D:2 * _D, :],
                        preferred_element_type=jnp.float32)
        y = y + jnp.dot(mn[:, sl], wpost[t, 2 * _D:3 * _D, :],
                        preferred_element_type=jnp.float32)
        y = y + jnp.dot(mx[:, sl], wpost[t, 3 * _D:4 * _D, :],
                        preferred_element_type=jnp.float32)
        ys.append(y)
    out = jnp.concatenate(ys, axis=1) + bpost
    out = (
        jnp.dot(out, wlin_ref[...], preferred_element_type=jnp.float32)
        + blin_ref[...]
    )
    out_ref[...] = out

    @pl.when(i == 0)
    def _():
        acc_ref[...] = jnp.zeros_like(acc_ref)

    acc_ref[0:1, :] += jnp.sum(out, axis=0, keepdims=True)
    acc_ref[1:2, :] += jnp.sum(out * out, axis=0, keepdims=True)

    @pl.when(i == pl.num_programs(0) - 1)
    def _():
        stats_ref[...] = acc_ref[...]


def _post(x, u, sx, se, deg, zmn, zmx, ws, wm, wpost, bpost, wlin, blin):
    return pl.pallas_call(
        _post_body,
        grid=(_N // _NT,),
        in_specs=[
            pl.BlockSpec((_NT, _D), lambda i: (i, 0)),
            pl.BlockSpec((_NT, _TD), lambda i: (i, 0)),
            pl.BlockSpec((_NT, _D), lambda i: (i, 0)),
            pl.BlockSpec((_NT, _DE), lambda i: (i, 0)),
            pl.BlockSpec((_NT, 1), lambda i: (i, 0)),
            pl.BlockSpec((_NT, _TD), lambda i: (i, 0)),
            pl.BlockSpec((_NT, _TD), lambda i: (i, 0)),
            pl.BlockSpec((_D, _TD), lambda i: (0, 0)),
            pl.BlockSpec((_DE, _TD), lambda i: (0, 0)),
            pl.BlockSpec((_T, 4 * _D, _FOUT), lambda i: (0, 0, 0)),
            pl.BlockSpec((1, _D), lambda i: (0, 0)),
            pl.BlockSpec((_D, _D), lambda i: (0, 0)),
            pl.BlockSpec((1, _D), lambda i: (0, 0)),
        ],
        out_specs=[
            pl.BlockSpec((_NT, _D), lambda i: (i, 0)),
            pl.BlockSpec((8, _D), lambda i: (0, 0)),
        ],
        out_shape=[
            jax.ShapeDtypeStruct((_N, _D), jnp.float32),
            jax.ShapeDtypeStruct((8, _D), jnp.float32),
        ],
        scratch_shapes=[pltpu.VMEM((8, _D), jnp.float32)],
    )(x, u, sx, se, deg, zmn, zmx, ws, wm, wpost, bpost, wlin, blin)


def _bn_body(raw_ref, stats_ref, gamma_ref, beta_ref, out_ref):
    s = stats_ref[...]
    mu = s[0:1, :] / _N
    var = s[1:2, :] / _N - mu * mu
    rstd = lax.rsqrt(var + 1e-5)
    out = (raw_ref[...] - mu) * rstd * gamma_ref[...] + beta_ref[...]
    out_ref[...] = jnp.where(out >= 0.0, out, 0.01 * out)


def _bn(raw, stats, gamma, beta):
    return pl.pallas_call(
        _bn_body,
        grid=(_N // _NT,),
        in_specs=[
            pl.BlockSpec((_NT, _D), lambda i: (i, 0)),
            pl.BlockSpec((8, _D), lambda i: (0, 0)),
            pl.BlockSpec((1, _D), lambda i: (0, 0)),
            pl.BlockSpec((1, _D), lambda i: (0, 0)),
        ],
        out_specs=pl.BlockSpec((_NT, _D), lambda i: (i, 0)),
        out_shape=jax.ShapeDtypeStruct((_N, _D), jnp.float32),
    )(raw, stats, gamma, beta)


def _s2s_body(h0_ref, h1_ref, batch_ref, wih0_ref, whh0_ref, b0_ref,
              wih1_ref, whh1_ref, b1_ref, out_ref):
    x = jnp.maximum(h0_ref[...], h1_ref[...])
    bt = batch_ref[...]
    iota = lax.broadcasted_iota(jnp.int32, (_N, _B), 1)
    onehotb = bt == iota
    onehot = onehotb.astype(jnp.float32)

    q_star = jnp.zeros((_B, 2 * _D), jnp.float32)
    h0 = jnp.zeros((_B, _D), jnp.float32)
    c0 = jnp.zeros((_B, _D), jnp.float32)
    h1 = jnp.zeros((_B, _D), jnp.float32)
    c1 = jnp.zeros((_B, _D), jnp.float32)

    def cell(xin, h, c, wih, whh, b):
        g = (
            lax.dot_general(xin, wih, (((1,), (1,)), ((), ())),
                            preferred_element_type=jnp.float32)
            + lax.dot_general(h, whh, (((1,), (1,)), ((), ())),
                              preferred_element_type=jnp.float32)
            + b
        )
        i = jax.nn.sigmoid(g[:, 0:_D])
        f = jax.nn.sigmoid(g[:, _D:2 * _D])
        gg = jnp.tanh(g[:, 2 * _D:3 * _D])
        o = jax.nn.sigmoid(g[:, 3 * _D:4 * _D])
        c2 = f * c + i * gg
        return o * jnp.tanh(c2), c2

    for _ in range(_STEPS):
        h0, c0 = cell(q_star, h0, c0, wih0_ref[...], whh0_ref[...], b0_ref[...])
        h1, c1 = cell(h0, h1, c1, wih1_ref[...], whh1_ref[...], b1_ref[...])
        q = h1
        qb = jnp.dot(onehot, q, preferred_element_type=jnp.float32)
        e = jnp.sum(x * qb, axis=1, keepdims=True)
        m = jnp.max(jnp.where(onehotb, e, -jnp.inf), axis=0, keepdims=True)
        m = jnp.where(jnp.isfinite(m), m, 0.0)
        mb = jnp.sum(onehot * m, axis=1, keepdims=True)
        ex = jnp.exp(e - mb)
        s = jnp.sum(onehot * ex, axis=0, keepdims=True)
        sb = jnp.sum(onehot * s, axis=1, keepdims=True)
        a = ex / (sb + 1e-16)
        r = lax.dot_general(onehot * a, x, (((0,), (0,)), ((), ())),
                            preferred_element_type=jnp.float32)
        q_star = jnp.concatenate([q, r], axis=1)
    out_ref[...] = q_star


def _s2s(h0, h1, batch2d, wih0, whh0, b0, wih1, whh1, b1):
    return pl.pallas_call(
        _s2s_body,
        out_shape=jax.ShapeDtypeStruct((_B, 2 * _D), jnp.float32),
    )(h0, h1, batch2d, wih0, whh0, b0, wih1, whh1, b1)


@jax.jit
def kernel(x, edge_index, edge_attr, batch, We, be, Wpre, bpre, Wpost, bpost,
           Wlin, blin, gamma, beta, Wih0, Whh0, bih0, bhh0, Wih1, Whh1,
           bih1, bhh1):
    src = edge_index[0]
    dst = edge_index[1]
    ones = jnp.ones((_E,), jnp.float32)
    deg = jax.ops.segment_sum(ones, dst, num_segments=_N)
    deg2d = deg[:, None]
    se = jax.ops.segment_sum(edge_attr, dst, num_segments=_N)

    h = x
    layer_outs = []
    for l in range(_L):
        wd = jnp.transpose(Wpre[l, :, 0:_D, :], (1, 0, 2)).reshape(_D, _TD)
        ws = jnp.transpose(Wpre[l, :, _D:2 * _D, :], (1, 0, 2)).reshape(_D, _TD)
        wein = jnp.transpose(Wpre[l, :, 2 * _D:3 * _D, :], (1, 0, 2)).reshape(_D, _TD)
        wm = We[l] @ wein
        ub = bpre[l].reshape(_TD) + be[l] @ wein
        wc = jnp.concatenate([wd, ws], axis=1)
        bu = jnp.concatenate([ub, jnp.zeros((_TD,), jnp.float32)])[None, :]

        uv = _uv(h, wc, bu)
        u = uv[:, 0:_TD]
        v = uv[:, _TD:]
        w = _edge_w(edge_attr, wm)

        z = v[src] + w
        zcat = jnp.concatenate([z, -z], axis=1)
        zext = jax.ops.segment_min(zcat, dst, num_segments=_N)
        zmn = zext[:, :_TD]
        zmx = -zext[:, _TD:]
        sx = jax.ops.segment_sum(h[src], dst, num_segments=_N)

        raw, stats = _post(
            h, u, sx, se, deg2d, zmn, zmx, ws, wm, Wpost[l],
            bpost[l].reshape(1, _T * _FOUT), Wlin[l], blin[l][None, :])
        h = _bn(raw, stats, gamma[l][None, :], beta[l][None, :])
        layer_outs.append(h)

    return _s2s(layer_outs[0], layer_outs[1], batch[:, None],
                Wih0, Whh0, (bih0 + bhh0)[None, :],
                Wih1, Whh1, (bih1 + bhh1)[None, :])


# speedup vs baseline: 14.4665x; 1.0072x over previous
import functools

import jax
import jax.numpy as jnp
from jax import lax
from jax.experimental import pallas as pl
from jax.experimental.pallas import tpu as pltpu

_L = 2
_N = 10000
_E = 160000
_D = 128
_DE = 16
_T = 4
_FOUT = 32
_B = 64
_STEPS = 5
_TD = _T * _D

_NT = 2000
_ET = 2000


def _uv_body(h_ref, wc_ref, bu_ref, u_ref, v_ref):
    uv = (
        jnp.dot(h_ref[...], wc_ref[...], preferred_element_type=jnp.float32)
        + bu_ref[...]
    )
    u_ref[...] = uv[:, 0:_TD]
    v_ref[...] = uv[:, _TD:]


def _uv(h, wc, bu):
    return pl.pallas_call(
        _uv_body,
        grid=(_N // _NT,),
        in_specs=[
            pl.BlockSpec((_NT, _D), lambda i: (i, 0)),
            pl.BlockSpec((_D, 2 * _TD), lambda i: (0, 0)),
            pl.BlockSpec((1, 2 * _TD), lambda i: (0, 0)),
        ],
        out_specs=[
            pl.BlockSpec((_NT, _TD), lambda i: (i, 0)),
            pl.BlockSpec((_NT, _TD), lambda i: (i, 0)),
        ],
        out_shape=[
            jax.ShapeDtypeStruct((_N, _TD), jnp.float32),
            jax.ShapeDtypeStruct((_N, _TD), jnp.float32),
        ],
    )(h, wc, bu)


def _w_body(ea_ref, wm_ref, w_ref):
    w_ref[...] = jnp.dot(
        ea_ref[...], wm_ref[...], preferred_element_type=jnp.float32
    )


def _edge_w(edge_attr, wm):
    return pl.pallas_call(
        _w_body,
        grid=(_E // _ET,),
        in_specs=[
            pl.BlockSpec((_ET, _DE), lambda i: (i, 0)),
            pl.BlockSpec((_DE, _TD), lambda i: (0, 0)),
        ],
        out_specs=pl.BlockSpec((_ET, _TD), lambda i: (i, 0)),
        out_shape=jax.ShapeDtypeStruct((_E, _TD), jnp.float32),
    )(edge_attr, wm)


def _post_body(x_ref, u_ref, sx_ref, se_ref, deg_ref, zmn_ref, zmx_ref,
               ws_ref, wm_ref, wpost_ref, bpost_ref, wlin_ref, blin_ref,
               out_ref, stats_ref, acc_ref):
    i = pl.program_id(0)
    x = x_ref[...]
    u = u_ref[...]
    deg = deg_ref[...]
    ssum = (
        deg * u
        + jnp.dot(sx_ref[...], ws_ref[...], preferred_element_type=jnp.float32)
        + jnp.dot(se_ref[...], wm_ref[...], preferred_element_type=jnp.float32)
    )
    mean = ssum / jnp.maximum(deg, 1.0)
    has = deg > 0.0
    mn = jnp.where(has, u + zmn_ref[...], 0.0)
    mx = jnp.where(has, u + zmx_ref[...], 0.0)
    wpost = wpost_ref[...]
    bpost = bpost_ref[...]
    ys = []
    for t in range(_T):
        sl = slice(t * _D, (t + 1) * _D)
        y = jnp.dot(x, wpost[t, 0:_D, :], preferred_element_type=jnp.float32)
        y = y + jnp.dot(mean[:, sl], wpost[t, _D:2 * _D, :],
                        preferred_element_type=jnp.float32)
        y = y + jnp.dot(mn[:, sl], wpost[t, 2 * _D:3 * _D, :],
                        preferred_element_type=jnp.float32)
        y = y + jnp.dot(mx[:, sl], wpost[t, 3 * _D:4 * _D, :],
                        preferred_element_type=jnp.float32)
        ys.append(y)
    out = jnp.concatenate(ys, axis=1) + bpost
    out = (
        jnp.dot(out, wlin_ref[...], preferred_element_type=jnp.float32)
        + blin_ref[...]
    )
    out_ref[...] = out

    @pl.when(i == 0)
    def _():
        acc_ref[...] = jnp.zeros_like(acc_ref)

    acc_ref[0:1, :] += jnp.sum(out, axis=0, keepdims=True)
    acc_ref[1:2, :] += jnp.sum(out * out, axis=0, keepdims=True)

    @pl.when(i == pl.num_programs(0) - 1)
    def _():
        stats_ref[...] = acc_ref[...]


def _post(x, u, sx, se, deg, zmn, zmx, ws, wm, wpost, bpost, wlin, blin):
    return pl.pallas_call(
        _post_body,
        grid=(_N // _NT,),
        in_specs=[
            pl.BlockSpec((_NT, _D), lambda i: (i, 0)),
            pl.BlockSpec((_NT, _TD), lambda i: (i, 0)),
            pl.BlockSpec((_NT, _D), lambda i: (i, 0)),
            pl.BlockSpec((_NT, _DE), lambda i: (i, 0)),
            pl.BlockSpec((_NT, 1), lambda i: (i, 0)),
            pl.BlockSpec((_NT, _TD), lambda i: (i, 0)),
            pl.BlockSpec((_NT, _TD), lambda i: (i, 0)),
            pl.BlockSpec((_D, _TD), lambda i: (0, 0)),
            pl.BlockSpec((_DE, _TD), lambda i: (0, 0)),
            pl.BlockSpec((_T, 4 * _D, _FOUT), lambda i: (0, 0, 0)),
            pl.BlockSpec((1, _D), lambda i: (0, 0)),
            pl.BlockSpec((_D, _D), lambda i: (0, 0)),
            pl.BlockSpec((1, _D), lambda i: (0, 0)),
        ],
        out_specs=[
            pl.BlockSpec((_NT, _D), lambda i: (i, 0)),
            pl.BlockSpec((8, _D), lambda i: (0, 0)),
        ],
        out_shape=[
            jax.ShapeDtypeStruct((_N, _D), jnp.float32),
            jax.ShapeDtypeStruct((8, _D), jnp.float32),
        ],
        scratch_shapes=[pltpu.VMEM((8, _D), jnp.float32)],
    )(x, u, sx, se, deg, zmn, zmx, ws, wm, wpost, bpost, wlin, blin)


def _bn_body(raw_ref, stats_ref, gamma_ref, beta_ref, out_ref):
    s = stats_ref[...]
    mu = s[0:1, :] / _N
    var = s[1:2, :] / _N - mu * mu
    rstd = lax.rsqrt(var + 1e-5)
    out = (raw_ref[...] - mu) * rstd * gamma_ref[...] + beta_ref[...]
    out_ref[...] = jnp.where(out >= 0.0, out, 0.01 * out)


def _bn(raw, stats, gamma, beta):
    return pl.pallas_call(
        _bn_body,
        grid=(_N // _NT,),
        in_specs=[
            pl.BlockSpec((_NT, _D), lambda i: (i, 0)),
            pl.BlockSpec((8, _D), lambda i: (0, 0)),
            pl.BlockSpec((1, _D), lambda i: (0, 0)),
            pl.BlockSpec((1, _D), lambda i: (0, 0)),
        ],
        out_specs=pl.BlockSpec((_NT, _D), lambda i: (i, 0)),
        out_shape=jax.ShapeDtypeStruct((_N, _D), jnp.float32),
    )(raw, stats, gamma, beta)


def _s2s_body(h0_ref, h1_ref, batch_ref, wih0_ref, whh0_ref, b0_ref,
              wih1_ref, whh1_ref, b1_ref, out_ref):
    x = jnp.maximum(h0_ref[...], h1_ref[...])
    bt = batch_ref[...]
    iota = lax.broadcasted_iota(jnp.int32, (_N, _B), 1)
    onehotb = bt == iota
    onehot = onehotb.astype(jnp.float32)

    q_star = jnp.zeros((_B, 2 * _D), jnp.float32)
    h0 = jnp.zeros((_B, _D), jnp.float32)
    c0 = jnp.zeros((_B, _D), jnp.float32)
    h1 = jnp.zeros((_B, _D), jnp.float32)
    c1 = jnp.zeros((_B, _D), jnp.float32)

    def cell(xin, h, c, wih, whh, b):
        g = (
            lax.dot_general(xin, wih, (((1,), (1,)), ((), ())),
                            preferred_element_type=jnp.float32)
            + lax.dot_general(h, whh, (((1,), (1,)), ((), ())),
                              preferred_element_type=jnp.float32)
            + b
        )
        i = jax.nn.sigmoid(g[:, 0:_D])
        f = jax.nn.sigmoid(g[:, _D:2 * _D])
        gg = jnp.tanh(g[:, 2 * _D:3 * _D])
        o = jax.nn.sigmoid(g[:, 3 * _D:4 * _D])
        c2 = f * c + i * gg
        return o * jnp.tanh(c2), c2

    for _ in range(_STEPS):
        h0, c0 = cell(q_star, h0, c0, wih0_ref[...], whh0_ref[...], b0_ref[...])
        h1, c1 = cell(h0, h1, c1, wih1_ref[...], whh1_ref[...], b1_ref[...])
        q = h1
        qb = jnp.dot(onehot, q, preferred_element_type=jnp.float32)
        e = jnp.sum(x * qb, axis=1, keepdims=True)
        m = jnp.max(jnp.where(onehotb, e, -jnp.inf), axis=0, keepdims=True)
        m = jnp.where(jnp.isfinite(m), m, 0.0)
        mb = jnp.sum(onehot * m, axis=1, keepdims=True)
        ex = jnp.exp(e - mb)
        s = jnp.sum(onehot * ex, axis=0, keepdims=True)
        sb = jnp.sum(onehot * s, axis=1, keepdims=True)
        a = ex / (sb + 1e-16)
        r = lax.dot_general(onehot * a, x, (((0,), (0,)), ((), ())),
                            preferred_element_type=jnp.float32)
        q_star = jnp.concatenate([q, r], axis=1)
    out_ref[...] = q_star


def _s2s(h0, h1, batch2d, wih0, whh0, b0, wih1, whh1, b1):
    return pl.pallas_call(
        _s2s_body,
        out_shape=jax.ShapeDtypeStruct((_B, 2 * _D), jnp.float32),
    )(h0, h1, batch2d, wih0, whh0, b0, wih1, whh1, b1)


@jax.jit
def kernel(x, edge_index, edge_attr, batch, We, be, Wpre, bpre, Wpost, bpost,
           Wlin, blin, gamma, beta, Wih0, Whh0, bih0, bhh0, Wih1, Whh1,
           bih1, bhh1):
    src = edge_index[0]
    dst = edge_index[1]
    ones = jnp.ones((_E,), jnp.float32)
    deg = jax.ops.segment_sum(ones, dst, num_segments=_N)
    deg2d = deg[:, None]
    se = jax.ops.segment_sum(edge_attr, dst, num_segments=_N)

    h = x
    layer_outs = []
    for l in range(_L):
        wd = jnp.transpose(Wpre[l, :, 0:_D, :], (1, 0, 2)).reshape(_D, _TD)
        ws = jnp.transpose(Wpre[l, :, _D:2 * _D, :], (1, 0, 2)).reshape(_D, _TD)
        wein = jnp.transpose(Wpre[l, :, 2 * _D:3 * _D, :], (1, 0, 2)).reshape(_D, _TD)
        wm = We[l] @ wein
        ub = bpre[l].reshape(_TD) + be[l] @ wein
        wc = jnp.concatenate([wd, ws], axis=1)
        bu = jnp.concatenate([ub, jnp.zeros((_TD,), jnp.float32)])[None, :]

        u, v = _uv(h, wc, bu)
        w = _edge_w(edge_attr, wm)

        z = v[src] + w
        zcat = jnp.concatenate([z, -z], axis=1)
        zext = jax.ops.segment_min(zcat, dst, num_segments=_N)
        zmn = zext[:, :_TD]
        zmx = -zext[:, _TD:]
        sx = jax.ops.segment_sum(h[src], dst, num_segments=_N)

        raw, stats = _post(
            h, u, sx, se, deg2d, zmn, zmx, ws, wm, Wpost[l],
            bpost[l].reshape(1, _T * _FOUT), Wlin[l], blin[l][None, :])
        h = _bn(raw, stats, gamma[l][None, :], beta[l][None, :])
        layer_outs.append(h)

    return _s2s(layer_outs[0], layer_outs[1], batch[:, None],
                Wih0, Whh0, (bih0 + bhh0)[None, :],
                Wih1, Whh1, (bih1 + bhh1)[None, :])
